# baseline (speedup 1.0000x reference)
import functools

import jax
import jax.numpy as jnp
from jax import lax
from jax.experimental import pallas as pl
from jax.experimental.pallas import tpu as pltpu
from jax.experimental.pallas import tpu_sc as plsc

NC = 2
NS = 16
NW = NC * NS
CH = 80
SCH = 72
RPT = 640
NPAD = NS * RPT
LANES = 16
BLK = 1280


def _sc_mesh():
    return plsc.VectorSubcoreMesh(core_axis_name="c", subcore_axis_name="s")


def _make_degree_kernel(n_chunks):
    @functools.partial(
        pl.kernel,
        out_type=jax.ShapeDtypeStruct((NC, NPAD), jnp.float32),
        mesh=_sc_mesh(),
        scratch_types=[
            pltpu.VMEM((n_chunks, CH), jnp.int32),
            pltpu.VMEM((CH,), jnp.float32),
            pltpu.VMEM((RPT,), jnp.float32),
            pltpu.VMEM_SHARED((NPAD,), jnp.float32),
        ],
    )
    def degree_kernel(dst_hbm, out_hbm, dst_v, ones_v, zero_v, acc_sh):
        cid = lax.axis_index("c")
        sid = lax.axis_index("s")
        wid = cid * NS + sid
        base = sid * RPT

        def fill(i, _):
            ones_v[pl.ds(i * LANES, LANES)] = jnp.ones((LANES,), jnp.float32)
            return 0

        lax.fori_loop(0, CH // LANES, fill, 0)

        def fillz(i, _):
            zero_v[pl.ds(i * LANES, LANES)] = jnp.zeros((LANES,), jnp.float32)
            return 0

        lax.fori_loop(0, RPT // LANES, fillz, 0)
        pltpu.sync_copy(zero_v, acc_sh.at[pl.ds(base, RPT)])
        plsc.subcore_barrier()

        pltpu.sync_copy(dst_hbm.at[wid], dst_v)

        def step(j, _):
            pltpu.sync_copy(ones_v, acc_sh.at[dst_v.at[j]], add=True)
            return 0

        lax.fori_loop(0, n_chunks, step, 0)
        plsc.subcore_barrier()
        pltpu.sync_copy(acc_sh.at[pl.ds(base, RPT)],
                        out_hbm.at[cid, pl.ds(base, RPT)])

    return degree_kernel


def _make_scatter_kernel(n_chunks, d):
    n_groups = 5
    gch = n_chunks // n_groups

    @functools.partial(
        pl.kernel,
        out_type=jax.ShapeDtypeStruct((NC, NPAD, d), jnp.float32),
        mesh=_sc_mesh(),
        scratch_types=[
            pltpu.VMEM((gch, SCH), jnp.int32),
            pltpu.VMEM((gch, SCH), jnp.int32),
            [pltpu.VMEM((SCH, d), jnp.float32) for _ in range(3)],
            pltpu.VMEM((32, d), jnp.float32),
            pltpu.VMEM_SHARED((NPAD, d), jnp.float32),
            [pltpu.SemaphoreType.DMA for _ in range(3)],
            [pltpu.SemaphoreType.DMA for _ in range(3)],
        ],
    )
    def scatter_kernel(hs_hbm, src_hbm, dst_hbm, out_hbm,
                       src_v, dst_v, rows, zero_v, acc_sh, gsem, ssem):
        cid = lax.axis_index("c")
        sid = lax.axis_index("s")
        wid = cid * NS + sid
        base = sid * RPT

        def gath(j, b):
            pltpu.async_copy(hs_hbm.at[src_v.at[j]], rows[b], gsem[b])

        def gath_wait(j, b):
            pltpu.make_async_copy(hs_hbm.at[src_v.at[j]], rows[b],
                                  gsem[b]).wait()

        def scat(j, b):
            pltpu.async_copy(rows[b], acc_sh.at[dst_v.at[j]], ssem[b],
                             add=True)

        def scat_wait(j, b):
            pltpu.make_async_copy(rows[b], acc_sh.at[dst_v.at[j]],
                                  ssem[b]).wait()

        pltpu.sync_copy(src_hbm.at[wid, 0], src_v)
        pltpu.sync_copy(dst_hbm.at[wid, 0], dst_v)
        gath(0, 0)
        gath(1, 1)
        gath(2, 2)

        def fz(i, _):
            r = i // (d // LANES)
            c = lax.rem(i, d // LANES) * LANES
            zero_v[r, pl.ds(c, LANES)] = jnp.zeros((LANES,), jnp.float32)
            return 0

        lax.fori_loop(0, 32 * (d // LANES), fz, 0)

        def zc(k, _):
            pltpu.sync_copy(zero_v, acc_sh.at[pl.ds(base + k * 32, 32)])
            return 0

        lax.fori_loop(0, RPT // 32, zc, 0)
        plsc.subcore_barrier()

        def consume():
            gath_wait(0, 0)
            scat(0, 0)
            gath_wait(1, 1)
            scat(1, 1)

            def ring(u, _):
                j = 2 + 3 * u
                scat_wait(j - 2, 0)
                gath(j + 1, 0)
                gath_wait(j, 2)
                scat(j, 2)
                scat_wait(j - 1, 1)
                gath(j + 2, 1)
                gath_wait(j + 1, 0)
                scat(j + 1, 0)
                scat_wait(j, 2)
                gath(j + 3, 2)
                gath_wait(j + 2, 1)
                scat(j + 2, 1)
                return 0

            lax.fori_loop(0, (gch - 4) // 3, ring, 0)
            scat_wait(gch - 4, 0)
            gath(gch - 1, 0)
            gath_wait(gch - 2, 2)
            scat(gch - 2, 2)
            gath_wait(gch - 1, 0)
            scat(gch - 1, 0)
            scat_wait(gch - 3, 1)
            scat_wait(gch - 2, 2)
            scat_wait(gch - 1, 0)

        consume()

        def group(g, _):
            pltpu.sync_copy(src_hbm.at[wid, g], src_v)
            pltpu.sync_copy(dst_hbm.at[wid, g], dst_v)
            gath(0, 0)
            gath(1, 1)
            gath(2, 2)
            consume()
            return 0

        lax.fori_loop(1, n_groups, group, 0)
        plsc.subcore_barrier()
        pltpu.sync_copy(acc_sh.at[pl.ds(base, RPT)],
                        out_hbm.at[cid, pl.ds(base, RPT)])

    return scatter_kernel


def _tc_first_body(x_ref, d0_ref, d1_ref, w_ref, dinv_ref, hs_ref):
    deg = d0_ref[...] + d1_ref[...] + 1.0
    dinv = lax.rsqrt(deg)
    dinv_ref[...] = dinv
    h = jnp.dot(x_ref[...], w_ref[...], preferred_element_type=jnp.float32)
    hs_ref[...] = h * dinv


def _tc_mid_body(a0_ref, a1_ref, hs_ref, dinv_ref, b_ref, g_ref, bt_ref,
                 w_ref, out_ref):
    dinv = dinv_ref[...]
    s = (a0_ref[...] + a1_ref[...] + hs_ref[...]) * dinv + b_ref[...]
    g = jnp.maximum(s, 0.0)
    mu = jnp.mean(g, axis=1, keepdims=True)
    var = jnp.mean((g - mu) ** 2, axis=1, keepdims=True)
    ln = (g - mu) * lax.rsqrt(var + 1e-5) * g_ref[...] + bt_ref[...]
    out_ref[...] = jnp.dot(ln, w_ref[...],
                           preferred_element_type=jnp.float32) * dinv


def _tc_last_body(a0_ref, a1_ref, hs_ref, dinv_ref, b_ref, g_ref, bt_ref,
                  wc_ref, bc_ref, out_ref):
    dinv = dinv_ref[...]
    s = (a0_ref[...] + a1_ref[...] + hs_ref[...]) * dinv + b_ref[...]
    g = jnp.maximum(s, 0.0)
    mu = jnp.mean(g, axis=1, keepdims=True)
    var = jnp.mean((g - mu) ** 2, axis=1, keepdims=True)
    ln = (g - mu) * lax.rsqrt(var + 1e-5) * g_ref[...] + bt_ref[...]
    out_ref[...] = jnp.dot(ln, wc_ref[...],
                           preferred_element_type=jnp.float32) + bc_ref[...]


def _row_spec(d):
    return pl.BlockSpec((BLK, d), lambda i: (i, 0))


def _full_spec(shape):
    return pl.BlockSpec(shape, lambda i: tuple(0 for _ in shape))


def kernel(x, edge_index, W1, b1, W2, b2, W3, b3, gamma, beta, Wc, bc):
    n, d_in = x.shape
    e = edge_index.shape[1]
    hidden = W1.shape[1]
    out_d = Wc.shape[1]
    epw = e // NW
    n_chunks = epw // CH
    grid = (NPAD // BLK,)

    n_groups = 5
    gsz = n_groups * SCH
    epw_pad = ((epw + gsz - 1) // gsz) * gsz
    s_chunks = epw_pad // SCH
    src = jnp.pad(edge_index[0].reshape(NW, epw),
                  ((0, 0), (0, epw_pad - epw)))
    dst = jnp.pad(edge_index[1].reshape(NW, epw),
                  ((0, 0), (0, epw_pad - epw)), constant_values=NPAD - 1)
    src = src.reshape(NW, n_groups, s_chunks // n_groups, SCH)
    dst = dst.reshape(NW, n_groups, s_chunks // n_groups, SCH)
    dst_flat = edge_index[1].reshape(NW, n_chunks, CH)

    degp = _make_degree_kernel(n_chunks)(dst_flat)
    d0 = degp[0].reshape(NPAD, 1)
    d1 = degp[1].reshape(NPAD, 1)

    blk_a = n // 10
    dinv_col, hs1 = pl.pallas_call(
        _tc_first_body,
        grid=(n // blk_a,),
        in_specs=[
            pl.BlockSpec((blk_a, d_in), lambda i: (i, 0)),
            pl.BlockSpec((blk_a, 1), lambda i: (i, 0)),
            pl.BlockSpec((blk_a, 1), lambda i: (i, 0)),
            _full_spec((d_in, hidden)),
        ],
        out_specs=[
            pl.BlockSpec((blk_a, 1), lambda i: (i, 0)),
            pl.BlockSpec((blk_a, hidden), lambda i: (i, 0)),
        ],
        out_shape=[
            jax.ShapeDtypeStruct((NPAD, 1), jnp.float32),
            jax.ShapeDtypeStruct((NPAD, hidden), jnp.float32),
        ],
    )(x, d0, d1, W1)

    scatter = _make_scatter_kernel(s_chunks, hidden)

    def mid_layer(hs, b_l, w_next):
        acc = scatter(hs, src, dst)
        return pl.pallas_call(
            _tc_mid_body,
            grid=grid,
            in_specs=[
                _row_spec(hidden), _row_spec(hidden), _row_spec(hidden),
                _row_spec(1),
                _full_spec((1, hidden)), _full_spec((1, hidden)),
                _full_spec((1, hidden)), _full_spec((hidden, hidden)),
            ],
            out_specs=_row_spec(hidden),
            out_shape=jax.ShapeDtypeStruct((NPAD, hidden), jnp.float32),
        )(acc[0], acc[1], hs, dinv_col, b_l.reshape(1, hidden),
          gamma.reshape(1, hidden), beta.reshape(1, hidden), w_next)

    hs2 = mid_layer(hs1, b1, W2)
    hs3 = mid_layer(hs2, b2, W3)

    acc3 = scatter(hs3, src, dst)
    out = pl.pallas_call(
        _tc_last_body,
        grid=grid,
        in_specs=[
            _row_spec(hidden), _row_spec(hidden), _row_spec(hidden),
            _row_spec(1),
            _full_spec((1, hidden)), _full_spec((1, hidden)),
            _full_spec((1, hidden)), _full_spec((hidden, out_d)),
            _full_spec((1, out_d)),
        ],
        out_specs=_row_spec(out_d),
        out_shape=jax.ShapeDtypeStruct((NPAD, out_d), jnp.float32),
    )(acc3[0], acc3[1], hs3, dinv_col, b3.reshape(1, hidden),
      gamma.reshape(1, hidden), beta.reshape(1, hidden), Wc,
      bc.reshape(1, out_d))

    return out[:n]


# speedup vs baseline: 1.5740x; 1.5740x over previous
import functools

import jax
import jax.numpy as jnp
from jax import lax
from jax.experimental import pallas as pl
from jax.experimental.pallas import tpu as pltpu
from jax.experimental.pallas import tpu_sc as plsc

NC = 2
NS = 16
NW = NC * NS
CH = 80
SCH = 80
RPT = 640
NPAD = NS * RPT
LANES = 16
BLK = 1280


def _sc_mesh():
    return plsc.VectorSubcoreMesh(core_axis_name="c", subcore_axis_name="s")


def _make_degree_kernel(n_chunks):
    @functools.partial(
        pl.kernel,
        out_type=jax.ShapeDtypeStruct((NC, NPAD), jnp.float32),
        mesh=_sc_mesh(),
        scratch_types=[
            pltpu.VMEM((n_chunks, CH), jnp.int32),
            pltpu.VMEM((CH,), jnp.float32),
            pltpu.VMEM((RPT,), jnp.float32),
            pltpu.VMEM_SHARED((NPAD,), jnp.float32),
        ],
    )
    def degree_kernel(dst_hbm, out_hbm, dst_v, ones_v, zero_v, acc_sh):
        cid = lax.axis_index("c")
        sid = lax.axis_index("s")
        wid = cid * NS + sid
        base = sid * RPT

        def fill(i, _):
            ones_v[pl.ds(i * LANES, LANES)] = jnp.ones((LANES,), jnp.float32)
            return 0

        lax.fori_loop(0, CH // LANES, fill, 0)

        def fillz(i, _):
            zero_v[pl.ds(i * LANES, LANES)] = jnp.zeros((LANES,), jnp.float32)
            return 0

        lax.fori_loop(0, RPT // LANES, fillz, 0)
        pltpu.sync_copy(zero_v, acc_sh.at[pl.ds(base, RPT)])
        plsc.subcore_barrier()

        pltpu.sync_copy(dst_hbm.at[wid], dst_v)

        def step(j, _):
            pltpu.sync_copy(ones_v, acc_sh.at[dst_v.at[j]], add=True)
            return 0

        lax.fori_loop(0, n_chunks, step, 0)
        plsc.subcore_barrier()
        pltpu.sync_copy(acc_sh.at[pl.ds(base, RPT)],
                        out_hbm.at[cid, pl.ds(base, RPT)])

    return degree_kernel


def _make_scatter_kernel(n_chunks, d):
    n_groups = 5
    gch = n_chunks // n_groups

    @functools.partial(
        pl.kernel,
        out_type=jax.ShapeDtypeStruct((NC, NPAD, d), jnp.float32),
        mesh=_sc_mesh(),
        scratch_types=[
            pltpu.VMEM((gch, SCH), jnp.int32),
            pltpu.VMEM((gch, SCH), jnp.int32),
            [pltpu.VMEM((SCH, d), jnp.float32) for _ in range(3)],
            pltpu.VMEM((8, d), jnp.float32),
            pltpu.VMEM_SHARED((NPAD, d), jnp.float32),
            [pltpu.SemaphoreType.DMA for _ in range(3)],
            [pltpu.SemaphoreType.DMA for _ in range(3)],
        ],
    )
    def scatter_kernel(hs_hbm, src_hbm, dst_hbm, out_hbm,
                       src_v, dst_v, rows, zero_v, acc_sh, gsem, ssem):
        cid = lax.axis_index("c")
        sid = lax.axis_index("s")
        wid = cid * NS + sid
        base = sid * RPT

        def gath(j, b):
            pltpu.async_copy(hs_hbm.at[src_v.at[j]], rows[b], gsem[b])

        def gath_wait(j, b):
            pltpu.make_async_copy(hs_hbm.at[src_v.at[j]], rows[b],
                                  gsem[b]).wait()

        def scat(j, b):
            pltpu.async_copy(rows[b], acc_sh.at[dst_v.at[j]], ssem[b],
                             add=True)

        def scat_wait(j, b):
            pltpu.make_async_copy(rows[b], acc_sh.at[dst_v.at[j]],
                                  ssem[b]).wait()

        pltpu.sync_copy(src_hbm.at[wid, 0], src_v)
        pltpu.sync_copy(dst_hbm.at[wid, 0], dst_v)
        gath(0, 0)
        gath(1, 1)
        gath(2, 2)

        def fz(i, _):
            r = i // (d // LANES)
            c = lax.rem(i, d // LANES) * LANES
            zero_v[r, pl.ds(c, LANES)] = jnp.zeros((LANES,), jnp.float32)
            return 0

        lax.fori_loop(0, 8 * (d // LANES), fz, 0)

        def zc(k, _):
            pltpu.sync_copy(zero_v, acc_sh.at[pl.ds(base + k * 8, 8)])
            return 0

        lax.fori_loop(0, RPT // 8, zc, 0)
        plsc.subcore_barrier()

        def consume():
            gath_wait(0, 0)
            scat(0, 0)
            gath_wait(1, 1)
            scat(1, 1)

            def ring(u, _):
                j = 2 + 3 * u
                scat_wait(j - 2, 0)
                gath(j + 1, 0)
                gath_wait(j, 2)
                scat(j, 2)
                scat_wait(j - 1, 1)
                gath(j + 2, 1)
                gath_wait(j + 1, 0)
                scat(j + 1, 0)
                scat_wait(j, 2)
                gath(j + 3, 2)
                gath_wait(j + 2, 1)
                scat(j + 2, 1)
                return 0

            lax.fori_loop(0, (gch - 4) // 3, ring, 0)
            scat_wait(gch - 4, 0)
            gath(gch - 1, 0)
            gath_wait(gch - 2, 2)
            scat(gch - 2, 2)
            gath_wait(gch - 1, 0)
            scat(gch - 1, 0)
            scat_wait(gch - 3, 1)
            scat_wait(gch - 2, 2)
            scat_wait(gch - 1, 0)

        consume()

        def group(g, _):
            pltpu.sync_copy(src_hbm.at[wid, g], src_v)
            pltpu.sync_copy(dst_hbm.at[wid, g], dst_v)
            gath(0, 0)
            gath(1, 1)
            gath(2, 2)
            consume()
            return 0

        lax.fori_loop(1, n_groups, group, 0)
        plsc.subcore_barrier()
        pltpu.sync_copy(acc_sh.at[pl.ds(base, RPT)],
                        out_hbm.at[cid, pl.ds(base, RPT)])

    return scatter_kernel


def _tc_first_body(x_ref, d0_ref, d1_ref, w_ref, dinv_ref, hs_ref):
    deg = d0_ref[...] + d1_ref[...] + 1.0
    dinv = lax.rsqrt(deg)
    dinv_ref[...] = dinv
    h = jnp.dot(x_ref[...], w_ref[...], preferred_element_type=jnp.float32)
    hs_ref[...] = h * dinv


def _tc_mid_body(a0_ref, a1_ref, hs_ref, dinv_ref, b_ref, g_ref, bt_ref,
                 w_ref, out_ref):
    dinv = dinv_ref[...]
    s = (a0_ref[...] + a1_ref[...] + hs_ref[...]) * dinv + b_ref[...]
    g = jnp.maximum(s, 0.0)
    mu = jnp.mean(g, axis=1, keepdims=True)
    var = jnp.mean((g - mu) ** 2, axis=1, keepdims=True)
    ln = (g - mu) * lax.rsqrt(var + 1e-5) * g_ref[...] + bt_ref[...]
    out_ref[...] = jnp.dot(ln, w_ref[...],
                           preferred_element_type=jnp.float32) * dinv


def _tc_last_body(a0_ref, a1_ref, hs_ref, dinv_ref, b_ref, g_ref, bt_ref,
                  wc_ref, bc_ref, out_ref):
    dinv = dinv_ref[...]
    s = (a0_ref[...] + a1_ref[...] + hs_ref[...]) * dinv + b_ref[...]
    g = jnp.maximum(s, 0.0)
    mu = jnp.mean(g, axis=1, keepdims=True)
    var = jnp.mean((g - mu) ** 2, axis=1, keepdims=True)
    ln = (g - mu) * lax.rsqrt(var + 1e-5) * g_ref[...] + bt_ref[...]
    out_ref[...] = jnp.dot(ln, wc_ref[...],
                           preferred_element_type=jnp.float32) + bc_ref[...]


def _row_spec(d):
    return pl.BlockSpec((BLK, d), lambda i: (i, 0))


def _full_spec(shape):
    return pl.BlockSpec(shape, lambda i: tuple(0 for _ in shape))


def kernel(x, edge_index, W1, b1, W2, b2, W3, b3, gamma, beta, Wc, bc):
    n, d_in = x.shape
    e = edge_index.shape[1]
    hidden = W1.shape[1]
    out_d = Wc.shape[1]
    epw = e // NW
    n_chunks = epw // CH
    grid = (NPAD // BLK,)

    n_groups = 5
    gsz = n_groups * SCH
    epw_pad = ((epw + gsz - 1) // gsz) * gsz
    s_chunks = epw_pad // SCH
    src = jnp.pad(edge_index[0].reshape(NW, epw),
                  ((0, 0), (0, epw_pad - epw)))
    dst = jnp.pad(edge_index[1].reshape(NW, epw),
                  ((0, 0), (0, epw_pad - epw)), constant_values=NPAD - 1)
    src = src.reshape(NW, n_groups, s_chunks // n_groups, SCH)
    dst = dst.reshape(NW, n_groups, s_chunks // n_groups, SCH)
    dst_flat = edge_index[1].reshape(NW, n_chunks, CH)

    degp = _make_degree_kernel(n_chunks)(dst_flat)
    d0 = degp[0].reshape(NPAD, 1)
    d1 = degp[1].reshape(NPAD, 1)

    blk_a = n // 10
    dinv_col, hs1 = pl.pallas_call(
        _tc_first_body,
        grid=(n // blk_a,),
        in_specs=[
            pl.BlockSpec((blk_a, d_in), lambda i: (i, 0)),
            pl.BlockSpec((blk_a, 1), lambda i: (i, 0)),
            pl.BlockSpec((blk_a, 1), lambda i: (i, 0)),
            _full_spec((d_in, hidden)),
        ],
        out_specs=[
            pl.BlockSpec((blk_a, 1), lambda i: (i, 0)),
            pl.BlockSpec((blk_a, hidden), lambda i: (i, 0)),
        ],
        out_shape=[
            jax.ShapeDtypeStruct((NPAD, 1), jnp.float32),
            jax.ShapeDtypeStruct((NPAD, hidden), jnp.float32),
        ],
    )(x, d0, d1, W1)

    scatter = _make_scatter_kernel(s_chunks, hidden)

    def mid_layer(hs, b_l, w_next):
        acc = scatter(hs, src, dst)
        return pl.pallas_call(
            _tc_mid_body,
            grid=grid,
            in_specs=[
                _row_spec(hidden), _row_spec(hidden), _row_spec(hidden),
                _row_spec(1),
                _full_spec((1, hidden)), _full_spec((1, hidden)),
                _full_spec((1, hidden)), _full_spec((hidden, hidden)),
            ],
            out_specs=_row_spec(hidden),
            out_shape=jax.ShapeDtypeStruct((NPAD, hidden), jnp.float32),
        )(acc[0], acc[1], hs, dinv_col, b_l.reshape(1, hidden),
          gamma.reshape(1, hidden), beta.reshape(1, hidden), w_next)

    hs2 = mid_layer(hs1, b1, W2)
    hs3 = mid_layer(hs2, b2, W3)

    acc3 = scatter(hs3, src, dst)
    out = pl.pallas_call(
        _tc_last_body,
        grid=grid,
        in_specs=[
            _row_spec(hidden), _row_spec(hidden), _row_spec(hidden),
            _row_spec(1),
            _full_spec((1, hidden)), _full_spec((1, hidden)),
            _full_spec((1, hidden)), _full_spec((hidden, out_d)),
            _full_spec((1, out_d)),
        ],
        out_specs=_row_spec(out_d),
        out_shape=jax.ShapeDtypeStruct((NPAD, out_d), jnp.float32),
    )(acc3[0], acc3[1], hs3, dinv_col, b3.reshape(1, hidden),
      gamma.reshape(1, hidden), beta.reshape(1, hidden), Wc,
      bc.reshape(1, out_d))

    return out[:n]


# speedup vs baseline: 1.6079x; 1.0216x over previous
import functools

import jax
import jax.numpy as jnp
from jax import lax
from jax.experimental import pallas as pl
from jax.experimental.pallas import tpu as pltpu
from jax.experimental.pallas import tpu_sc as plsc

NC = 2
NS = 16
NW = NC * NS
CH = 80
SCH = 80
RPT = 640
NPAD = NS * RPT
LANES = 16
BLK = 1280


def _sc_mesh():
    return plsc.VectorSubcoreMesh(core_axis_name="c", subcore_axis_name="s")


def _make_degree_kernel(n_chunks):
    @functools.partial(
        pl.kernel,
        out_type=jax.ShapeDtypeStruct((NC, NPAD), jnp.float32),
        mesh=_sc_mesh(),
        scratch_types=[
            pltpu.VMEM((n_chunks, CH), jnp.int32),
            pltpu.VMEM((CH,), jnp.float32),
            pltpu.VMEM((RPT,), jnp.float32),
            pltpu.VMEM_SHARED((NPAD,), jnp.float32),
        ],
    )
    def degree_kernel(dst_hbm, out_hbm, dst_v, ones_v, zero_v, acc_sh):
        cid = lax.axis_index("c")
        sid = lax.axis_index("s")
        wid = cid * NS + sid
        base = sid * RPT

        def fill(i, _):
            ones_v[pl.ds(i * LANES, LANES)] = jnp.ones((LANES,), jnp.float32)
            return 0

        lax.fori_loop(0, CH // LANES, fill, 0)

        def fillz(i, _):
            zero_v[pl.ds(i * LANES, LANES)] = jnp.zeros((LANES,), jnp.float32)
            return 0

        lax.fori_loop(0, RPT // LANES, fillz, 0)
        pltpu.sync_copy(zero_v, acc_sh.at[pl.ds(base, RPT)])
        plsc.subcore_barrier()

        pltpu.sync_copy(dst_hbm.at[wid], dst_v)

        def step(j, _):
            pltpu.sync_copy(ones_v, acc_sh.at[dst_v.at[j]], add=True)
            return 0

        lax.fori_loop(0, n_chunks, step, 0)
        plsc.subcore_barrier()
        pltpu.sync_copy(acc_sh.at[pl.ds(base, RPT)],
                        out_hbm.at[cid, pl.ds(base, RPT)])

    return degree_kernel


def _make_scatter_kernel(n_chunks, d):
    n_groups = 5
    gch = n_chunks // n_groups

    @functools.partial(
        pl.kernel,
        out_type=jax.ShapeDtypeStruct((NC, NPAD, d), jnp.float32),
        mesh=_sc_mesh(),
        scratch_types=[
            pltpu.VMEM((gch, SCH), jnp.int32),
            pltpu.VMEM((gch, SCH), jnp.int32),
            [pltpu.VMEM((SCH, d), jnp.float32) for _ in range(3)],
            pltpu.VMEM((8, d), jnp.float32),
            pltpu.VMEM_SHARED((NPAD, d), jnp.float32),
            [pltpu.SemaphoreType.DMA for _ in range(3)],
            [pltpu.SemaphoreType.DMA for _ in range(3)],
        ],
    )
    def scatter_kernel(hs_hbm, src_hbm, dst_hbm, out_hbm,
                       src_v, dst_v, rows, zero_v, acc_sh, gsem, ssem):
        cid = lax.axis_index("c")
        sid = lax.axis_index("s")
        wid = cid * NS + sid
        base = sid * RPT

        def gath(j, b):
            pltpu.async_copy(hs_hbm.at[src_v.at[j]], rows[b], gsem[b])

        def gath_wait(j, b):
            pltpu.make_async_copy(hs_hbm.at[src_v.at[j]], rows[b],
                                  gsem[b]).wait()

        def scat(j, b):
            pltpu.async_copy(rows[b], acc_sh.at[dst_v.at[j]], ssem[b],
                             add=True)

        def scat_wait(j, b):
            pltpu.make_async_copy(rows[b], acc_sh.at[dst_v.at[j]],
                                  ssem[b]).wait()

        def fz(i, _):
            r = i // (d // LANES)
            c = lax.rem(i, d // LANES) * LANES
            zero_v[r, pl.ds(c, LANES)] = jnp.zeros((LANES,), jnp.float32)
            return 0

        lax.fori_loop(0, 8 * (d // LANES), fz, 0)

        def zc(k, _):
            pltpu.sync_copy(zero_v, acc_sh.at[pl.ds(base + k * 8, 8)])
            return 0

        lax.fori_loop(0, RPT // 8, zc, 0)
        plsc.subcore_barrier()

        def group(g, _):
            pltpu.sync_copy(src_hbm.at[wid, g], src_v)
            pltpu.sync_copy(dst_hbm.at[wid, g], dst_v)
            gath(0, 0)
            gath(1, 1)
            gath_wait(0, 0)
            scat(0, 0)
            gath(2, 2)
            gath_wait(1, 1)
            scat(1, 1)

            def ring(u, _):
                j = 2 + 3 * u
                scat_wait(j - 2, 0)
                gath(j + 1, 0)
                gath_wait(j, 2)
                scat(j, 2)
                scat_wait(j - 1, 1)
                gath(j + 2, 1)
                gath_wait(j + 1, 0)
                scat(j + 1, 0)
                scat_wait(j, 2)
                gath(j + 3, 2)
                gath_wait(j + 2, 1)
                scat(j + 2, 1)
                return 0

            lax.fori_loop(0, (gch - 4) // 3, ring, 0)
            scat_wait(gch - 4, 0)
            gath(gch - 1, 0)
            gath_wait(gch - 2, 2)
            scat(gch - 2, 2)
            gath_wait(gch - 1, 0)
            scat(gch - 1, 0)
            scat_wait(gch - 3, 1)
            scat_wait(gch - 2, 2)
            scat_wait(gch - 1, 0)
            return 0

        lax.fori_loop(0, n_groups, group, 0)
        plsc.subcore_barrier()
        pltpu.sync_copy(acc_sh.at[pl.ds(base, RPT)],
                        out_hbm.at[cid, pl.ds(base, RPT)])

    return scatter_kernel


def _tc_first_body(x_ref, d0_ref, d1_ref, w_ref, dinv_ref, hs_ref):
    deg = d0_ref[...] + d1_ref[...] + 1.0
    dinv = lax.rsqrt(deg)
    dinv_ref[...] = dinv
    h = jnp.dot(x_ref[...], w_ref[...], preferred_element_type=jnp.float32)
    hs_ref[...] = h * dinv


def _tc_mid_body(a0_ref, a1_ref, hs_ref, dinv_ref, b_ref, g_ref, bt_ref,
                 w_ref, out_ref):
    dinv = dinv_ref[...]
    s = (a0_ref[...] + a1_ref[...] + hs_ref[...]) * dinv + b_ref[...]
    g = jnp.maximum(s, 0.0)
    mu = jnp.mean(g, axis=1, keepdims=True)
    var = jnp.mean((g - mu) ** 2, axis=1, keepdims=True)
    ln = (g - mu) * lax.rsqrt(var + 1e-5) * g_ref[...] + bt_ref[...]
    out_ref[...] = jnp.dot(ln, w_ref[...],
                           preferred_element_type=jnp.float32) * dinv


def _tc_last_body(a0_ref, a1_ref, hs_ref, dinv_ref, b_ref, g_ref, bt_ref,
                  wc_ref, bc_ref, out_ref):
    dinv = dinv_ref[...]
    s = (a0_ref[...] + a1_ref[...] + hs_ref[...]) * dinv + b_ref[...]
    g = jnp.maximum(s, 0.0)
    mu = jnp.mean(g, axis=1, keepdims=True)
    var = jnp.mean((g - mu) ** 2, axis=1, keepdims=True)
    ln = (g - mu) * lax.rsqrt(var + 1e-5) * g_ref[...] + bt_ref[...]
    out_ref[...] = jnp.dot(ln, wc_ref[...],
                           preferred_element_type=jnp.float32) + bc_ref[...]


def _row_spec(d):
    return pl.BlockSpec((BLK, d), lambda i: (i, 0))


def _full_spec(shape):
    return pl.BlockSpec(shape, lambda i: tuple(0 for _ in shape))


def kernel(x, edge_index, W1, b1, W2, b2, W3, b3, gamma, beta, Wc, bc):
    n, d_in = x.shape
    e = edge_index.shape[1]
    hidden = W1.shape[1]
    out_d = Wc.shape[1]
    epw = e // NW
    n_chunks = epw // CH
    grid = (NPAD // BLK,)

    n_groups = 5
    gsz = n_groups * SCH
    epw_pad = ((epw + gsz - 1) // gsz) * gsz
    s_chunks = epw_pad // SCH
    src = jnp.pad(edge_index[0].reshape(NW, epw),
                  ((0, 0), (0, epw_pad - epw)))
    dst = jnp.pad(edge_index[1].reshape(NW, epw),
                  ((0, 0), (0, epw_pad - epw)), constant_values=NPAD - 1)
    src = src.reshape(NW, n_groups, s_chunks // n_groups, SCH)
    dst = dst.reshape(NW, n_groups, s_chunks // n_groups, SCH)
    dst_flat = edge_index[1].reshape(NW, n_chunks, CH)
    x_pad = jnp.pad(x, ((0, NPAD - n), (0, 0)))

    degp = _make_degree_kernel(n_chunks)(dst_flat)
    d0 = degp[0].reshape(NPAD, 1)
    d1 = degp[1].reshape(NPAD, 1)

    dinv_col, hs1 = pl.pallas_call(
        _tc_first_body,
        grid=grid,
        in_specs=[
            _row_spec(d_in),
            _row_spec(1),
            _row_spec(1),
            _full_spec((d_in, hidden)),
        ],
        out_specs=[_row_spec(1), _row_spec(hidden)],
        out_shape=[
            jax.ShapeDtypeStruct((NPAD, 1), jnp.float32),
            jax.ShapeDtypeStruct((NPAD, hidden), jnp.float32),
        ],
    )(x_pad, d0, d1, W1)

    scatter = _make_scatter_kernel(s_chunks, hidden)

    def mid_layer(hs, b_l, w_next):
        acc = scatter(hs, src, dst)
        return pl.pallas_call(
            _tc_mid_body,
            grid=grid,
            in_specs=[
                _row_spec(hidden), _row_spec(hidden), _row_spec(hidden),
                _row_spec(1),
                _full_spec((1, hidden)), _full_spec((1, hidden)),
                _full_spec((1, hidden)), _full_spec((hidden, hidden)),
            ],
            out_specs=_row_spec(hidden),
            out_shape=jax.ShapeDtypeStruct((NPAD, hidden), jnp.float32),
        )(acc[0], acc[1], hs, dinv_col, b_l.reshape(1, hidden),
          gamma.reshape(1, hidden), beta.reshape(1, hidden), w_next)

    hs2 = mid_layer(hs1, b1, W2)
    hs3 = mid_layer(hs2, b2, W3)

    acc3 = scatter(hs3, src, dst)
    out = pl.pallas_call(
        _tc_last_body,
        grid=grid,
        in_specs=[
            _row_spec(hidden), _row_spec(hidden), _row_spec(hidden),
            _row_spec(1),
            _full_spec((1, hidden)), _full_spec((1, hidden)),
            _full_spec((1, hidden)), _full_spec((hidden, out_d)),
            _full_spec((1, out_d)),
        ],
        out_specs=_row_spec(out_d),
        out_shape=jax.ShapeDtypeStruct((NPAD, out_d), jnp.float32),
    )(acc3[0], acc3[1], hs3, dinv_col, b3.reshape(1, hidden),
      gamma.reshape(1, hidden), beta.reshape(1, hidden), Wc,
      bc.reshape(1, out_d))

    return out[:n]


# speedup vs baseline: 1.6285x; 1.0128x over previous
import functools

import jax
import jax.numpy as jnp
from jax import lax
from jax.experimental import pallas as pl
from jax.experimental.pallas import tpu as pltpu
from jax.experimental.pallas import tpu_sc as plsc

NC = 2
NS = 16
NW = NC * NS
CH = 80
SCH = 80
RPT = 640
NPAD = NS * RPT
LANES = 16
BLK = 1280


def _sc_mesh():
    return plsc.VectorSubcoreMesh(core_axis_name="c", subcore_axis_name="s")


def _make_degree_kernel(n_chunks):
    @functools.partial(
        pl.kernel,
        out_type=jax.ShapeDtypeStruct((NC, NPAD), jnp.float32),
        mesh=_sc_mesh(),
        scratch_types=[
            pltpu.VMEM((n_chunks, CH), jnp.int32),
            pltpu.VMEM((CH,), jnp.float32),
            pltpu.VMEM((RPT,), jnp.float32),
            pltpu.VMEM_SHARED((NPAD,), jnp.float32),
            pltpu.SemaphoreType.DMA,
        ],
    )
    def degree_kernel(dst_hbm, out_hbm, dst_v, ones_v, zero_v, acc_sh, dsem):
        cid = lax.axis_index("c")
        sid = lax.axis_index("s")
        wid = cid * NS + sid
        base = sid * RPT

        def fill(i, _):
            ones_v[pl.ds(i * LANES, LANES)] = jnp.ones((LANES,), jnp.float32)
            return 0

        lax.fori_loop(0, CH // LANES, fill, 0)

        def fillz(i, _):
            zero_v[pl.ds(i * LANES, LANES)] = jnp.zeros((LANES,), jnp.float32)
            return 0

        lax.fori_loop(0, RPT // LANES, fillz, 0)
        pltpu.sync_copy(zero_v, acc_sh.at[pl.ds(base, RPT)])
        plsc.subcore_barrier()

        pltpu.sync_copy(dst_hbm.at[wid], dst_v)

        wave = n_chunks // 5

        def step(w, _):
            def fire(j, _):
                pltpu.async_copy(ones_v, acc_sh.at[dst_v.at[w * wave + j]],
                                 dsem, add=True)
                return 0

            lax.fori_loop(0, wave, fire, 0)

            def drain(j, _):
                pltpu.make_async_copy(ones_v, acc_sh.at[dst_v.at[0]],
                                      dsem).wait()
                return 0

            lax.fori_loop(0, wave, drain, 0)
            return 0

        lax.fori_loop(0, 5, step, 0)
        plsc.subcore_barrier()
        pltpu.sync_copy(acc_sh.at[pl.ds(base, RPT)],
                        out_hbm.at[cid, pl.ds(base, RPT)])

    return degree_kernel


def _make_scatter_kernel(n_chunks, d):
    n_groups = 5
    gch = n_chunks // n_groups

    @functools.partial(
        pl.kernel,
        out_type=jax.ShapeDtypeStruct((NC, NPAD, d), jnp.float32),
        mesh=_sc_mesh(),
        scratch_types=[
            pltpu.VMEM((gch, SCH), jnp.int32),
            pltpu.VMEM((gch, SCH), jnp.int32),
            [pltpu.VMEM((SCH, d), jnp.float32) for _ in range(3)],
            pltpu.VMEM((8, d), jnp.float32),
            pltpu.VMEM_SHARED((NPAD, d), jnp.float32),
            [pltpu.SemaphoreType.DMA for _ in range(3)],
            [pltpu.SemaphoreType.DMA for _ in range(3)],
        ],
    )
    def scatter_kernel(hs_hbm, src_hbm, dst_hbm, out_hbm,
                       src_v, dst_v, rows, zero_v, acc_sh, gsem, ssem):
        cid = lax.axis_index("c")
        sid = lax.axis_index("s")
        wid = cid * NS + sid
        base = sid * RPT

        def gath(j, b):
            pltpu.async_copy(hs_hbm.at[src_v.at[j]], rows[b], gsem[b])

        def gath_wait(j, b):
            pltpu.make_async_copy(hs_hbm.at[src_v.at[j]], rows[b],
                                  gsem[b]).wait()

        def scat(j, b):
            pltpu.async_copy(rows[b], acc_sh.at[dst_v.at[j]], ssem[b],
                             add=True)

        def scat_wait(j, b):
            pltpu.make_async_copy(rows[b], acc_sh.at[dst_v.at[j]],
                                  ssem[b]).wait()

        def fz(i, _):
            r = i // (d // LANES)
            c = lax.rem(i, d // LANES) * LANES
            zero_v[r, pl.ds(c, LANES)] = jnp.zeros((LANES,), jnp.float32)
            return 0

        lax.fori_loop(0, 8 * (d // LANES), fz, 0)

        def zc(k, _):
            pltpu.sync_copy(zero_v, acc_sh.at[pl.ds(base + k * 8, 8)])
            return 0

        lax.fori_loop(0, RPT // 8, zc, 0)
        plsc.subcore_barrier()

        def group(g, _):
            pltpu.sync_copy(src_hbm.at[wid, g], src_v)
            pltpu.sync_copy(dst_hbm.at[wid, g], dst_v)
            gath(0, 0)
            gath(1, 1)
            gath_wait(0, 0)
            scat(0, 0)
            gath(2, 2)
            gath_wait(1, 1)
            scat(1, 1)

            def ring(u, _):
                j = 2 + 3 * u
                scat_wait(j - 2, 0)
                gath(j + 1, 0)
                gath_wait(j, 2)
                scat(j, 2)
                scat_wait(j - 1, 1)
                gath(j + 2, 1)
                gath_wait(j + 1, 0)
                scat(j + 1, 0)
                scat_wait(j, 2)
                gath(j + 3, 2)
                gath_wait(j + 2, 1)
                scat(j + 2, 1)
                return 0

            lax.fori_loop(0, (gch - 4) // 3, ring, 0)
            scat_wait(gch - 4, 0)
            gath(gch - 1, 0)
            gath_wait(gch - 2, 2)
            scat(gch - 2, 2)
            gath_wait(gch - 1, 0)
            scat(gch - 1, 0)
            scat_wait(gch - 3, 1)
            scat_wait(gch - 2, 2)
            scat_wait(gch - 1, 0)
            return 0

        lax.fori_loop(0, n_groups, group, 0)
        plsc.subcore_barrier()
        pltpu.sync_copy(acc_sh.at[pl.ds(base, RPT)],
                        out_hbm.at[cid, pl.ds(base, RPT)])

    return scatter_kernel


def _tc_first_body(x_ref, d0_ref, d1_ref, w_ref, dinv_ref, hs_ref):
    deg = d0_ref[...] + d1_ref[...] + 1.0
    dinv = lax.rsqrt(deg)
    dinv_ref[...] = dinv
    h = jnp.dot(x_ref[...], w_ref[...], preferred_element_type=jnp.float32)
    hs_ref[...] = h * dinv


def _tc_mid_body(a0_ref, a1_ref, hs_ref, dinv_ref, b_ref, g_ref, bt_ref,
                 w_ref, out_ref):
    dinv = dinv_ref[...]
    s = (a0_ref[...] + a1_ref[...] + hs_ref[...]) * dinv + b_ref[...]
    g = jnp.maximum(s, 0.0)
    mu = jnp.mean(g, axis=1, keepdims=True)
    var = jnp.mean((g - mu) ** 2, axis=1, keepdims=True)
    ln = (g - mu) * lax.rsqrt(var + 1e-5) * g_ref[...] + bt_ref[...]
    out_ref[...] = jnp.dot(ln, w_ref[...],
                           preferred_element_type=jnp.float32) * dinv


def _tc_last_body(a0_ref, a1_ref, hs_ref, dinv_ref, b_ref, g_ref, bt_ref,
                  wc_ref, bc_ref, out_ref):
    dinv = dinv_ref[...]
    s = (a0_ref[...] + a1_ref[...] + hs_ref[...]) * dinv + b_ref[...]
    g = jnp.maximum(s, 0.0)
    mu = jnp.mean(g, axis=1, keepdims=True)
    var = jnp.mean((g - mu) ** 2, axis=1, keepdims=True)
    ln = (g - mu) * lax.rsqrt(var + 1e-5) * g_ref[...] + bt_ref[...]
    out_ref[...] = jnp.dot(ln, wc_ref[...],
                           preferred_element_type=jnp.float32) + bc_ref[...]


def _row_spec(d):
    return pl.BlockSpec((BLK, d), lambda i: (i, 0))


def _full_spec(shape):
    return pl.BlockSpec(shape, lambda i: tuple(0 for _ in shape))


def kernel(x, edge_index, W1, b1, W2, b2, W3, b3, gamma, beta, Wc, bc):
    n, d_in = x.shape
    e = edge_index.shape[1]
    hidden = W1.shape[1]
    out_d = Wc.shape[1]
    epw = e // NW
    n_chunks = epw // CH
    grid = (NPAD // BLK,)

    n_groups = 5
    gsz = n_groups * SCH
    epw_pad = ((epw + gsz - 1) // gsz) * gsz
    s_chunks = epw_pad // SCH
    src = jnp.pad(edge_index[0].reshape(NW, epw),
                  ((0, 0), (0, epw_pad - epw)))
    dst = jnp.pad(edge_index[1].reshape(NW, epw),
                  ((0, 0), (0, epw_pad - epw)), constant_values=NPAD - 1)
    src = src.reshape(NW, n_groups, s_chunks // n_groups, SCH)
    dst = dst.reshape(NW, n_groups, s_chunks // n_groups, SCH)
    dst_flat = edge_index[1].reshape(NW, n_chunks, CH)
    x_pad = jnp.pad(x, ((0, NPAD - n), (0, 0)))

    degp = _make_degree_kernel(n_chunks)(dst_flat)
    d0 = degp[0].reshape(NPAD, 1)
    d1 = degp[1].reshape(NPAD, 1)

    dinv_col, hs1 = pl.pallas_call(
        _tc_first_body,
        grid=grid,
        in_specs=[
            _row_spec(d_in),
            _row_spec(1),
            _row_spec(1),
            _full_spec((d_in, hidden)),
        ],
        out_specs=[_row_spec(1), _row_spec(hidden)],
        out_shape=[
            jax.ShapeDtypeStruct((NPAD, 1), jnp.float32),
            jax.ShapeDtypeStruct((NPAD, hidden), jnp.float32),
        ],
    )(x_pad, d0, d1, W1)

    scatter = _make_scatter_kernel(s_chunks, hidden)

    def mid_layer(hs, b_l, w_next):
        acc = scatter(hs, src, dst)
        return pl.pallas_call(
            _tc_mid_body,
            grid=grid,
            in_specs=[
                _row_spec(hidden), _row_spec(hidden), _row_spec(hidden),
                _row_spec(1),
                _full_spec((1, hidden)), _full_spec((1, hidden)),
                _full_spec((1, hidden)), _full_spec((hidden, hidden)),
            ],
            out_specs=_row_spec(hidden),
            out_shape=jax.ShapeDtypeStruct((NPAD, hidden), jnp.float32),
        )(acc[0], acc[1], hs, dinv_col, b_l.reshape(1, hidden),
          gamma.reshape(1, hidden), beta.reshape(1, hidden), w_next)

    hs2 = mid_layer(hs1, b1, W2)
    hs3 = mid_layer(hs2, b2, W3)

    acc3 = scatter(hs3, src, dst)
    out = pl.pallas_call(
        _tc_last_body,
        grid=grid,
        in_specs=[
            _row_spec(hidden), _row_spec(hidden), _row_spec(hidden),
            _row_spec(1),
            _full_spec((1, hidden)), _full_spec((1, hidden)),
            _full_spec((1, hidden)), _full_spec((hidden, out_d)),
            _full_spec((1, out_d)),
        ],
        out_specs=_row_spec(out_d),
        out_shape=jax.ShapeDtypeStruct((NPAD, out_d), jnp.float32),
    )(acc3[0], acc3[1], hs3, dinv_col, b3.reshape(1, hidden),
      gamma.reshape(1, hidden), beta.reshape(1, hidden), Wc,
      bc.reshape(1, out_d))

    return out[:n]
